# baseline (speedup 1.0000x reference)
import functools

import jax
import jax.numpy as jnp
from jax.experimental import pallas as pl
from jax.experimental.shard_map import shard_map
from jax.sharding import PartitionSpec as P


def _pick_bm(n_local):
    best = 8
    for d in range(8, 513, 8):
        if n_local % d == 0:
            best = d
    return best


def _xw_kernel(x_ref, w_ref, o_ref):
    o_ref[...] = jnp.dot(x_ref[...], w_ref[...].T,
                         preferred_element_type=jnp.float32)


def _pass_b_kernel(adj_ref, t_ref, b1_ref, w2_ref, u_ref):
    h = jnp.dot(adj_ref[...], t_ref[...], preferred_element_type=jnp.float32)
    h = jnp.maximum(h + b1_ref[...], 0.0)
    u_ref[...] = jnp.dot(h, w2_ref[...].T, preferred_element_type=jnp.float32)


def _pass_c_kernel(adj_ref, u_ref, b2_ref, o_ref):
    z = jnp.dot(adj_ref[...], u_ref[...], preferred_element_type=jnp.float32)
    z = z + b2_ref[...]
    m = jnp.max(z, axis=1, keepdims=True)
    e = z - m
    lse = jnp.log(jnp.sum(jnp.exp(e), axis=1, keepdims=True))
    o_ref[...] = e - lse


def _gcn_shard(x, adj_loc, W1, b1, W2, b2, *, n, axis_present):
    in_f = x.shape[1]
    hid = W1.shape[0]
    out_f = W2.shape[0]
    n_loc = adj_loc.shape[0]
    bm = _pick_bm(n_loc)
    grid = (n_loc // bm,)

    t = pl.pallas_call(
        _xw_kernel,
        out_shape=jax.ShapeDtypeStruct((n, hid), jnp.float32),
        in_specs=[
            pl.BlockSpec((n, in_f), lambda: (0, 0)),
            pl.BlockSpec((hid, in_f), lambda: (0, 0)),
        ],
        out_specs=pl.BlockSpec((n, hid), lambda: (0, 0)),
    )(x, W1)

    u_loc = pl.pallas_call(
        _pass_b_kernel,
        grid=grid,
        out_shape=jax.ShapeDtypeStruct((n_loc, out_f), jnp.float32),
        in_specs=[
            pl.BlockSpec((bm, n), lambda i: (i, 0)),
            pl.BlockSpec((n, hid), lambda i: (0, 0)),
            pl.BlockSpec((hid,), lambda i: (0,)),
            pl.BlockSpec((out_f, hid), lambda i: (0, 0)),
        ],
        out_specs=pl.BlockSpec((bm, out_f), lambda i: (i, 0)),
    )(adj_loc, t, b1, W2)

    if axis_present:
        u = jax.lax.all_gather(u_loc, "i", axis=0, tiled=True)
    else:
        u = u_loc

    out_loc = pl.pallas_call(
        _pass_c_kernel,
        grid=grid,
        out_shape=jax.ShapeDtypeStruct((n_loc, out_f), jnp.float32),
        in_specs=[
            pl.BlockSpec((bm, n), lambda i: (i, 0)),
            pl.BlockSpec((n, out_f), lambda i: (0, 0)),
            pl.BlockSpec((out_f,), lambda i: (0,)),
        ],
        out_specs=pl.BlockSpec((bm, out_f), lambda i: (i, 0)),
    )(adj_loc, u, b2)

    return out_loc


@jax.jit
def kernel(x, adj, W1, b1, W2, b2):
    n = adj.shape[0]
    devs = jax.devices()
    n_dev = len(devs)
    while n_dev > 1 and n % n_dev != 0:
        n_dev -= 1
    if n_dev > 1:
        mesh = jax.sharding.Mesh(devs[:n_dev], ("i",))
        fn = shard_map(
            functools.partial(_gcn_shard, n=n, axis_present=True),
            mesh=mesh,
            in_specs=(P(None, None), P("i", None), P(None, None), P(None),
                      P(None, None), P(None)),
            out_specs=P("i", None),
            check_rep=False,
        )
        return fn(x, adj, W1, b1, W2, b2)
    return _gcn_shard(x, adj, W1, b1, W2, b2, n=n, axis_present=False)


# speedup vs baseline: 2.8785x; 2.8785x over previous
import jax
import jax.numpy as jnp
from jax.experimental import pallas as pl

BM = 400


def _xw_kernel(x_ref, w_ref, o_ref):
    o_ref[...] = jnp.dot(x_ref[...], w_ref[...].T,
                         preferred_element_type=jnp.float32)


def _pass_b_kernel(adj_ref, t_ref, b1_ref, w2_ref, u_ref):
    h = jnp.dot(adj_ref[...].astype(jnp.bfloat16), t_ref[...],
                preferred_element_type=jnp.float32)
    h = jnp.maximum(h + b1_ref[...], 0.0)
    u_ref[...] = jnp.dot(h.astype(jnp.bfloat16), w2_ref[...].T,
                         preferred_element_type=jnp.float32)


def _pass_c_kernel(adj_ref, u_ref, b2_ref, o_ref):
    z = jnp.dot(adj_ref[...].astype(jnp.bfloat16), u_ref[...],
                preferred_element_type=jnp.float32)
    z = z + b2_ref[...]
    m = jnp.max(z, axis=1, keepdims=True)
    e = z - m
    lse = jnp.log(jnp.sum(jnp.exp(e), axis=1, keepdims=True))
    o_ref[...] = e - lse


@jax.jit
def kernel(x, adj, W1, b1, W2, b2):
    in_f = x.shape[1]
    hid = W1.shape[0]
    out_f = W2.shape[0]
    n = adj.shape[0]
    grid = (n // BM,)

    t = pl.pallas_call(
        _xw_kernel,
        out_shape=jax.ShapeDtypeStruct((n, hid), jnp.float32),
        in_specs=[
            pl.BlockSpec((n, in_f), lambda: (0, 0)),
            pl.BlockSpec((hid, in_f), lambda: (0, 0)),
        ],
        out_specs=pl.BlockSpec((n, hid), lambda: (0, 0)),
    )(x, W1)
    t16 = t.astype(jnp.bfloat16)

    u = pl.pallas_call(
        _pass_b_kernel,
        grid=grid,
        out_shape=jax.ShapeDtypeStruct((n, out_f), jnp.float32),
        in_specs=[
            pl.BlockSpec((BM, n), lambda i: (i, 0)),
            pl.BlockSpec((n, hid), lambda i: (0, 0)),
            pl.BlockSpec((hid,), lambda i: (0,)),
            pl.BlockSpec((out_f, hid), lambda i: (0, 0)),
        ],
        out_specs=pl.BlockSpec((BM, out_f), lambda i: (i, 0)),
    )(adj, t16, b1, W2.astype(jnp.bfloat16))

    out = pl.pallas_call(
        _pass_c_kernel,
        grid=grid,
        out_shape=jax.ShapeDtypeStruct((n, out_f), jnp.float32),
        in_specs=[
            pl.BlockSpec((BM, n), lambda i: (i, 0)),
            pl.BlockSpec((n, out_f), lambda i: (0, 0)),
            pl.BlockSpec((out_f,), lambda i: (0,)),
        ],
        out_specs=pl.BlockSpec((BM, out_f), lambda i: (i, 0)),
    )(adj, u.astype(jnp.bfloat16), b2)

    return out


# speedup vs baseline: 3.1024x; 1.0778x over previous
import jax
import jax.numpy as jnp
from jax.experimental import pallas as pl
from jax.experimental.pallas import tpu as pltpu

BM = 400


def _fused_kernel(x_ref, adj_ref, w1_ref, b1_ref, w2_ref, b2_ref,
                  out_ref, t_ref, u_ref):
    d = pl.program_id(0)
    i = pl.program_id(1)

    @pl.when((d == 0) & (i == 0))
    def _():
        t_ref[...] = jnp.dot(x_ref[...], w1_ref[...].T,
                             preferred_element_type=jnp.float32)

    @pl.when(d == 0)
    def _():
        h = jnp.dot(adj_ref[...], t_ref[...],
                    preferred_element_type=jnp.float32)
        h = jnp.maximum(h + b1_ref[...], 0.0)
        u_ref[pl.ds(i * BM, BM), :] = jnp.dot(
            h, w2_ref[...].T, preferred_element_type=jnp.float32)

    @pl.when(d == 1)
    def _():
        z = jnp.dot(adj_ref[...], u_ref[...],
                    preferred_element_type=jnp.float32)
        z = z + b2_ref[...]
        m = jnp.max(z, axis=1, keepdims=True)
        e = z - m
        lse = jnp.log(jnp.sum(jnp.exp(e), axis=1, keepdims=True))
        out_ref[...] = e - lse


@jax.jit
def kernel(x, adj, W1, b1, W2, b2):
    in_f = x.shape[1]
    hid = W1.shape[0]
    out_f = W2.shape[0]
    n = adj.shape[0]

    return pl.pallas_call(
        _fused_kernel,
        grid=(2, n // BM),
        out_shape=jax.ShapeDtypeStruct((n, out_f), jnp.float32),
        in_specs=[
            pl.BlockSpec((n, in_f), lambda d, i: (0, 0)),
            pl.BlockSpec((BM, n), lambda d, i: (i, 0)),
            pl.BlockSpec((hid, in_f), lambda d, i: (0, 0)),
            pl.BlockSpec((hid,), lambda d, i: (0,)),
            pl.BlockSpec((out_f, hid), lambda d, i: (0, 0)),
            pl.BlockSpec((out_f,), lambda d, i: (0,)),
        ],
        out_specs=pl.BlockSpec((BM, out_f), lambda d, i: (i, 0)),
        scratch_shapes=[
            pltpu.VMEM((n, hid), jnp.float32),
            pltpu.VMEM((n, out_f), jnp.float32),
        ],
    )(x, adj, W1, b1, W2, b2)
